# baseline (speedup 1.0000x reference)
import functools

import jax
import jax.numpy as jnp
from jax import lax
from jax.experimental import pallas as pl
from jax.experimental.pallas import tpu as pltpu
from jax.experimental.pallas import tpu_sc as plsc

N = 10000
CH = 128
E = 320000
NC = 2
NS = 16
NT = NC * NS
CHUNK = 128
JCH = 80
CPT0 = 96
CPT1 = 64
NCHUNKS = NS * (CPT0 + CPT1)
EPAD = NCHUNKS * CHUNK
TRASH = N
NROWS = 10112
RPT = NROWS // NS
DEGN = 10240
DPT = DEGN // NS

_mesh = plsc.VectorSubcoreMesh(
    core_axis_name="c", subcore_axis_name="s", num_cores=NC, num_subcores=NS
)


@functools.partial(
    pl.kernel,
    out_type=jax.ShapeDtypeStruct((NC, 1, DEGN), jnp.float32),
    mesh=_mesh,
    scratch_types=[
        pltpu.VMEM((JCH, CHUNK), jnp.int32),
        pltpu.VMEM((CHUNK,), jnp.float32),
        pltpu.VMEM((DPT,), jnp.float32),
        pltpu.VMEM_SHARED((DEGN,), jnp.float32),
    ],
)
def _deg_kernel(dst_hbm, out_hbm, dst_v, ones_v, zero_v, deg_sh):
    cid = lax.axis_index("c")
    sid = lax.axis_index("s")
    wid = cid * NS + sid

    zeros16 = jnp.zeros((16,), jnp.float32)
    ones16 = jnp.ones((16,), jnp.float32)

    def zfill(i, carry):
        zero_v[pl.ds(i * 16, 16)] = zeros16
        return carry

    lax.fori_loop(0, DPT // 16, zfill, 0)

    def ofill(i, carry):
        ones_v[pl.ds(i * 16, 16)] = ones16
        return carry

    lax.fori_loop(0, CHUNK // 16, ofill, 0)

    pltpu.sync_copy(zero_v, deg_sh.at[pl.ds(sid * DPT, DPT)])
    pltpu.sync_copy(dst_hbm.at[wid], dst_v)
    plsc.subcore_barrier()

    def body(j, carry):
        pltpu.sync_copy(ones_v, deg_sh.at[dst_v.at[j]], add=True)
        return carry

    lax.fori_loop(0, JCH, body, 0)
    plsc.subcore_barrier()

    pltpu.sync_copy(
        deg_sh.at[pl.ds(sid * DPT, DPT)], out_hbm.at[cid, 0, pl.ds(sid * DPT, DPT)]
    )


@functools.partial(
    pl.kernel,
    out_type=jax.ShapeDtypeStruct((NC, NROWS, CH), jnp.float32),
    mesh=_mesh,
    scratch_types=[
        pltpu.VMEM((CPT0, CHUNK), jnp.int32),
        pltpu.VMEM((CPT0, CHUNK), jnp.int32),
        pltpu.VMEM((CHUNK, CH), jnp.float32),
        pltpu.SemaphoreType.DMA,
        pltpu.VMEM_SHARED((NROWS, CH), jnp.float32),
    ],
)
def _agg_kernel(g_hbm, src_hbm, dst_hbm, out_hbm, src_v, dst_v, rows_a, sem_a, acc_sh):
    cid = lax.axis_index("c")
    sid = lax.axis_index("s")
    wid = cid * NS + sid

    zeros16 = jnp.zeros((16,), jnp.float32)

    def zfill(t, carry):
        i = t // (CH // 16)
        j = t % (CH // 16)
        rows_a[i, pl.ds(j * 16, 16)] = zeros16
        return carry

    lax.fori_loop(0, CHUNK * (CH // 16), zfill, 0)

    base = sid * RPT
    for k in range(RPT // CHUNK):
        pltpu.sync_copy(rows_a, acc_sh.at[pl.ds(base + k * CHUNK, CHUNK)])
    rem = RPT % CHUNK
    if rem:
        pltpu.sync_copy(
            rows_a.at[pl.ds(0, rem)],
            acc_sh.at[pl.ds(base + (RPT // CHUNK) * CHUNK, rem)],
        )

    @pl.when(cid == 0)
    def _():
        pltpu.sync_copy(src_hbm.at[pl.ds(sid * CPT0, CPT0)], src_v)
        pltpu.sync_copy(dst_hbm.at[pl.ds(sid * CPT0, CPT0)], dst_v)

    @pl.when(cid == 1)
    def _():
        pltpu.sync_copy(
            src_hbm.at[pl.ds(NS * CPT0 + sid * CPT1, CPT1)],
            src_v.at[pl.ds(0, CPT1)],
        )
        pltpu.sync_copy(
            dst_hbm.at[pl.ds(NS * CPT0 + sid * CPT1, CPT1)],
            dst_v.at[pl.ds(0, CPT1)],
        )

    plsc.subcore_barrier()

    def body(j, carry):
        pltpu.async_copy(g_hbm.at[src_v.at[j]], rows_a, sem_a).wait()
        pltpu.sync_copy(rows_a, acc_sh.at[dst_v.at[j]], add=True)
        return carry

    @pl.when(cid == 0)
    def _():
        lax.fori_loop(0, CPT0, body, 0)

    @pl.when(cid == 1)
    def _():
        lax.fori_loop(0, CPT1, body, 0)

    plsc.subcore_barrier()

    pltpu.sync_copy(
        acc_sh.at[pl.ds(base, RPT)], out_hbm.at[cid, pl.ds(base, RPT)]
    )


def _dense_body(x_ref, w_ref, degp_ref, g_ref):
    w = w_ref[...]
    cn = jnp.sqrt(jnp.sum(w * w, axis=0, keepdims=True))
    scale = jnp.where(cn > 1.0, 1.0 / cn, 1.0)
    xs = x_ref[...] * scale
    h = lax.dot_general(
        xs, w, (((1,), (1,)), ((), ())), preferred_element_type=jnp.float32
    )
    dp = degp_ref[...]
    dinv = 1.0 / jnp.sqrt(dp[:, 0:1] + dp[:, 1:2] + 1.0)
    g_ref[...] = h * dinv


_dense = pl.pallas_call(
    _dense_body,
    out_shape=jax.ShapeDtypeStruct((N, CH), jnp.float32),
)


def _combine_body(p_ref, g_ref, degp_ref, b_ref, o_ref):
    dp = degp_ref[...]
    dinv = 1.0 / jnp.sqrt(dp[:, 0:1] + dp[:, 1:2] + 1.0)
    s = p_ref[0] + p_ref[1] + g_ref[...]
    o_ref[...] = dinv * s + b_ref[...]


_combine = pl.pallas_call(
    _combine_body,
    out_shape=jax.ShapeDtypeStruct((N, CH), jnp.float32),
)


def kernel(x, edge_index, W, b):
    e = edge_index.astype(jnp.int32)
    src = e[0]
    dst = e[1]
    pad = EPAD - E
    src_p = jnp.concatenate([src, jnp.zeros((pad,), jnp.int32)])
    trash = TRASH + (jnp.arange(pad, dtype=jnp.int32) % (NROWS - N))
    dst_p = jnp.concatenate([dst, trash])
    dst3 = dst_p.reshape(NT, JCH, CHUNK)
    src2 = src_p.reshape(NCHUNKS, CHUNK)
    dst2 = dst_p.reshape(NCHUNKS, CHUNK)

    deg_parts = _deg_kernel(dst3)
    degp = deg_parts.reshape(NC, DEGN).T[:N]
    g = _dense(x, W, degp)
    parts = _agg_kernel(g, src2, dst2)
    out = _combine(parts[:, :N, :], g, degp, b.reshape(1, CH))
    return out


# speedup vs baseline: 2.2014x; 2.2014x over previous
import functools

import jax
import jax.numpy as jnp
from jax import lax
from jax.experimental import pallas as pl
from jax.experimental.pallas import tpu as pltpu
from jax.experimental.pallas import tpu_sc as plsc

N = 10000
CH = 128
E = 320000
NC = 2
NS = 16
NT = NC * NS
CHUNK = 128
JCH = 80
CPT0 = 96
CPT1 = 64
NCHUNKS = NS * (CPT0 + CPT1)
EPAD = NCHUNKS * CHUNK
TRASH = N
NROWS = 10112
RPT = NROWS // NS
DEGN = 10240
DPT = DEGN // NS

_mesh = plsc.VectorSubcoreMesh(
    core_axis_name="c", subcore_axis_name="s", num_cores=NC, num_subcores=NS
)


@functools.partial(
    pl.kernel,
    out_type=jax.ShapeDtypeStruct((NC, 1, DEGN), jnp.float32),
    mesh=_mesh,
    scratch_types=[
        pltpu.VMEM((JCH, CHUNK), jnp.int32),
        pltpu.VMEM((CHUNK,), jnp.float32),
        pltpu.VMEM((DPT,), jnp.float32),
        pltpu.VMEM_SHARED((DEGN,), jnp.float32),
    ],
)
def _deg_kernel(dst_hbm, out_hbm, dst_v, ones_v, zero_v, deg_sh):
    cid = lax.axis_index("c")
    sid = lax.axis_index("s")
    wid = cid * NS + sid

    zeros16 = jnp.zeros((16,), jnp.float32)
    ones16 = jnp.ones((16,), jnp.float32)

    def zfill(i, carry):
        zero_v[pl.ds(i * 16, 16)] = zeros16
        return carry

    lax.fori_loop(0, DPT // 16, zfill, 0)

    def ofill(i, carry):
        ones_v[pl.ds(i * 16, 16)] = ones16
        return carry

    lax.fori_loop(0, CHUNK // 16, ofill, 0)

    pltpu.sync_copy(zero_v, deg_sh.at[pl.ds(sid * DPT, DPT)])
    pltpu.sync_copy(dst_hbm.at[wid], dst_v)
    plsc.subcore_barrier()

    def body(j, carry):
        pltpu.sync_copy(ones_v, deg_sh.at[dst_v.at[j]], add=True)
        return carry

    lax.fori_loop(0, JCH, body, 0)
    plsc.subcore_barrier()

    pltpu.sync_copy(
        deg_sh.at[pl.ds(sid * DPT, DPT)], out_hbm.at[cid, 0, pl.ds(sid * DPT, DPT)]
    )


@functools.partial(
    pl.kernel,
    out_type=jax.ShapeDtypeStruct((NC, NROWS, CH), jnp.float32),
    mesh=_mesh,
    scratch_types=[
        pltpu.VMEM((CPT0, CHUNK), jnp.int32),
        pltpu.VMEM((CPT0, CHUNK), jnp.int32),
        pltpu.VMEM((CHUNK, CH), jnp.float32),
        pltpu.SemaphoreType.DMA,
        pltpu.VMEM_SHARED((NROWS, CH), jnp.float32),
    ],
)
def _agg_kernel(g_hbm, src_hbm, dst_hbm, out_hbm, src_v, dst_v, rows_a, sem_a, acc_sh):
    cid = lax.axis_index("c")
    sid = lax.axis_index("s")
    wid = cid * NS + sid

    zeros16 = jnp.zeros((16,), jnp.float32)

    def zfill(t, carry):
        i = t // (CH // 16)
        j = t % (CH // 16)
        rows_a[i, pl.ds(j * 16, 16)] = zeros16
        return carry

    lax.fori_loop(0, CHUNK * (CH // 16), zfill, 0)

    base = sid * RPT
    for k in range(RPT // CHUNK):
        pltpu.sync_copy(rows_a, acc_sh.at[pl.ds(base + k * CHUNK, CHUNK)])
    rem = RPT % CHUNK
    if rem:
        pltpu.sync_copy(
            rows_a.at[pl.ds(0, rem)],
            acc_sh.at[pl.ds(base + (RPT // CHUNK) * CHUNK, rem)],
        )

    @pl.when(cid == 0)
    def _():
        pltpu.sync_copy(src_hbm.at[pl.ds(sid * CPT0, CPT0)], src_v)
        pltpu.sync_copy(dst_hbm.at[pl.ds(sid * CPT0, CPT0)], dst_v)

    @pl.when(cid == 1)
    def _():
        pltpu.sync_copy(
            src_hbm.at[pl.ds(NS * CPT0 + sid * CPT1, CPT1)],
            src_v.at[pl.ds(0, CPT1)],
        )
        pltpu.sync_copy(
            dst_hbm.at[pl.ds(NS * CPT0 + sid * CPT1, CPT1)],
            dst_v.at[pl.ds(0, CPT1)],
        )

    plsc.subcore_barrier()

    def body(j, carry):
        pltpu.async_copy(g_hbm.at[src_v.at[j]], rows_a, sem_a).wait()
        pltpu.sync_copy(rows_a, acc_sh.at[dst_v.at[j]], add=True)
        return carry

    @pl.when(cid == 0)
    def _():
        lax.fori_loop(0, CPT0, body, 0)

    @pl.when(cid == 1)
    def _():
        lax.fori_loop(0, CPT1, body, 0)

    plsc.subcore_barrier()

    pltpu.sync_copy(
        acc_sh.at[pl.ds(base, RPT)], out_hbm.at[cid, pl.ds(base, RPT)]
    )


def _dense_body(x_ref, w_ref, degp_ref, g_ref):
    w = w_ref[...]
    cn = jnp.sqrt(jnp.sum(w * w, axis=0, keepdims=True))
    scale = jnp.where(cn > 1.0, 1.0 / cn, 1.0)
    xs = x_ref[...] * scale
    h = lax.dot_general(
        xs, w, (((1,), (1,)), ((), ())), preferred_element_type=jnp.float32
    )
    dp = degp_ref[...]
    dinv = 1.0 / jnp.sqrt(dp[:, 0:1] + dp[:, 1:2] + 1.0)
    g_ref[...] = h * dinv


_dense = pl.pallas_call(
    _dense_body,
    out_shape=jax.ShapeDtypeStruct((N, CH), jnp.float32),
)


def _combine_body(p_ref, g_ref, degp_ref, b_ref, o_ref):
    dp = degp_ref[...]
    dinv = 1.0 / jnp.sqrt(dp[:, 0:1] + dp[:, 1:2] + 1.0)
    s = p_ref[0] + p_ref[1] + g_ref[...]
    o_ref[...] = dinv * s + b_ref[...]


_combine = pl.pallas_call(
    _combine_body,
    out_shape=jax.ShapeDtypeStruct((N, CH), jnp.float32),
)


def kernel(x, edge_index, W, b):
    e = edge_index.astype(jnp.int32)
    src = e[0]
    dst = e[1]
    pad = EPAD - E
    ar = jnp.arange(pad, dtype=jnp.int32)
    src_p = jnp.concatenate([src, ar % N])
    dst_p = jnp.concatenate([dst, TRASH + (ar % (NROWS - N))])
    dst3 = dst_p.reshape(NT, JCH, CHUNK)
    src2 = src_p.reshape(NCHUNKS, CHUNK)
    dst2 = dst_p.reshape(NCHUNKS, CHUNK)

    deg_parts = _deg_kernel(dst3)
    degp = deg_parts.reshape(NC, DEGN).T[:N]
    g = _dense(x, W, degp)
    parts = _agg_kernel(g, src2, dst2)
    out = _combine(parts[:, :N, :], g, degp, b.reshape(1, CH))
    return out


# speedup vs baseline: 2.4670x; 1.1207x over previous
import functools

import jax
import jax.numpy as jnp
from jax import lax
from jax.experimental import pallas as pl
from jax.experimental.pallas import tpu as pltpu
from jax.experimental.pallas import tpu_sc as plsc

N = 10000
CH = 128
E = 320000
NC = 2
NS = 16
NT = NC * NS
CHUNK = 128
JCH = 80
EPAD = NT * JCH * CHUNK
TRASH = N
NROWS = 10112
RPT = NROWS // NS
DEGN = 10240
DPT = DEGN // NS

_mesh = plsc.VectorSubcoreMesh(
    core_axis_name="c", subcore_axis_name="s", num_cores=NC, num_subcores=NS
)


@functools.partial(
    pl.kernel,
    out_type=jax.ShapeDtypeStruct((NC, 1, DEGN), jnp.float32),
    mesh=_mesh,
    scratch_types=[
        pltpu.VMEM((JCH, CHUNK), jnp.int32),
        pltpu.VMEM((CHUNK,), jnp.float32),
        pltpu.VMEM((DPT,), jnp.float32),
        pltpu.VMEM_SHARED((DEGN,), jnp.float32),
    ],
)
def _deg_kernel(dst_hbm, out_hbm, dst_v, ones_v, zero_v, deg_sh):
    cid = lax.axis_index("c")
    sid = lax.axis_index("s")
    wid = cid * NS + sid

    zeros16 = jnp.zeros((16,), jnp.float32)
    ones16 = jnp.ones((16,), jnp.float32)

    def zfill(i, carry):
        zero_v[pl.ds(i * 16, 16)] = zeros16
        return carry

    lax.fori_loop(0, DPT // 16, zfill, 0)

    def ofill(i, carry):
        ones_v[pl.ds(i * 16, 16)] = ones16
        return carry

    lax.fori_loop(0, CHUNK // 16, ofill, 0)

    pltpu.sync_copy(zero_v, deg_sh.at[pl.ds(sid * DPT, DPT)])
    pltpu.sync_copy(dst_hbm.at[wid], dst_v)
    plsc.subcore_barrier()

    def body(j, carry):
        pltpu.sync_copy(ones_v, deg_sh.at[dst_v.at[j]], add=True)
        return carry

    lax.fori_loop(0, JCH, body, 0)
    plsc.subcore_barrier()

    pltpu.sync_copy(
        deg_sh.at[pl.ds(sid * DPT, DPT)], out_hbm.at[cid, 0, pl.ds(sid * DPT, DPT)]
    )


@functools.partial(
    pl.kernel,
    out_type=jax.ShapeDtypeStruct((NC, NROWS, CH), jnp.float32),
    mesh=_mesh,
    scratch_types=[
        pltpu.VMEM((JCH, CHUNK), jnp.int32),
        pltpu.VMEM((JCH, CHUNK), jnp.int32),
        pltpu.VMEM((CHUNK, CH), jnp.float32),
        pltpu.SemaphoreType.DMA,
        pltpu.VMEM_SHARED((NROWS, CH), jnp.float32),
    ],
)
def _agg_kernel(g_hbm, src_hbm, dst_hbm, out_hbm, src_v, dst_v, rows_a, sem_a, acc_sh):
    cid = lax.axis_index("c")
    sid = lax.axis_index("s")
    wid = cid * NS + sid

    zeros16 = jnp.zeros((16,), jnp.float32)

    def zfill(t, carry):
        i = t // (CH // 16)
        j = t % (CH // 16)
        rows_a[i, pl.ds(j * 16, 16)] = zeros16
        return carry

    lax.fori_loop(0, CHUNK * (CH // 16), zfill, 0)

    base = sid * RPT
    for k in range(RPT // CHUNK):
        pltpu.sync_copy(rows_a, acc_sh.at[pl.ds(base + k * CHUNK, CHUNK)])
    rem = RPT % CHUNK
    if rem:
        pltpu.sync_copy(
            rows_a.at[pl.ds(0, rem)],
            acc_sh.at[pl.ds(base + (RPT // CHUNK) * CHUNK, rem)],
        )

    pltpu.sync_copy(src_hbm.at[wid], src_v)
    pltpu.sync_copy(dst_hbm.at[wid], dst_v)
    plsc.subcore_barrier()

    def body(j, carry):
        pltpu.async_copy(g_hbm.at[src_v.at[j]], rows_a, sem_a).wait()
        pltpu.sync_copy(rows_a, acc_sh.at[dst_v.at[j]], add=True)
        return carry

    lax.fori_loop(0, JCH, body, 0)
    plsc.subcore_barrier()

    pltpu.sync_copy(
        acc_sh.at[pl.ds(base, RPT)], out_hbm.at[cid, pl.ds(base, RPT)]
    )


def _dense_body(x_ref, w_ref, degp_ref, g_ref):
    w = w_ref[...]
    cn = jnp.sqrt(jnp.sum(w * w, axis=0, keepdims=True))
    scale = jnp.where(cn > 1.0, 1.0 / cn, 1.0)
    xs = x_ref[...] * scale
    h = lax.dot_general(
        xs, w, (((1,), (1,)), ((), ())), preferred_element_type=jnp.float32
    )
    dp = degp_ref[...]
    dinv = 1.0 / jnp.sqrt(dp[:, 0:1] + dp[:, 1:2] + 1.0)
    g_ref[...] = h * dinv


_dense = pl.pallas_call(
    _dense_body,
    out_shape=jax.ShapeDtypeStruct((N, CH), jnp.float32),
)


def _combine_body(p_ref, g_ref, degp_ref, b_ref, o_ref):
    dp = degp_ref[...]
    dinv = 1.0 / jnp.sqrt(dp[:, 0:1] + dp[:, 1:2] + 1.0)
    s = p_ref[0] + p_ref[1] + g_ref[...]
    o_ref[...] = dinv * s + b_ref[...]


_combine = pl.pallas_call(
    _combine_body,
    out_shape=jax.ShapeDtypeStruct((N, CH), jnp.float32),
)


def kernel(x, edge_index, W, b):
    e = edge_index.astype(jnp.int32)
    src = e[0]
    dst = e[1]
    pad = EPAD - E
    ar = jnp.arange(pad, dtype=jnp.int32)
    src_p = jnp.concatenate([src, ar % N])
    dst_p = jnp.concatenate([dst, TRASH + (ar % (NROWS - N))])
    src3 = src_p.reshape(NT, JCH, CHUNK)
    dst3 = dst_p.reshape(NT, JCH, CHUNK)

    deg_parts = _deg_kernel(dst3)
    degp = deg_parts.reshape(NC, DEGN).T[:N]
    g = _dense(x, W, degp)
    parts = _agg_kernel(g, src3, dst3)
    out = _combine(parts[:, :N, :], g, degp, b.reshape(1, CH))
    return out


# speedup vs baseline: 2.5367x; 1.0282x over previous
import functools

import jax
import jax.numpy as jnp
from jax import lax
from jax.experimental import pallas as pl
from jax.experimental.pallas import tpu as pltpu
from jax.experimental.pallas import tpu_sc as plsc

N = 10000
CH = 128
E = 320000
NC = 2
NS = 16
NT = NC * NS
CHUNK = 128
JCH = 80
EPAD = NT * JCH * CHUNK
TRASH = N
NROWS = 10112
RPT = NROWS // NS
DEGN = 10240
DPT = DEGN // NS

_mesh = plsc.VectorSubcoreMesh(
    core_axis_name="c", subcore_axis_name="s", num_cores=NC, num_subcores=NS
)


@functools.partial(
    pl.kernel,
    out_type=jax.ShapeDtypeStruct((NC, 1, DEGN), jnp.float32),
    mesh=_mesh,
    scratch_types=[
        pltpu.VMEM((JCH, CHUNK), jnp.int32),
        pltpu.VMEM((CHUNK,), jnp.float32),
        pltpu.VMEM((DPT,), jnp.float32),
        pltpu.VMEM_SHARED((DEGN,), jnp.float32),
    ],
)
def _deg_kernel(dst_hbm, out_hbm, dst_v, ones_v, zero_v, deg_sh):
    cid = lax.axis_index("c")
    sid = lax.axis_index("s")
    wid = cid * NS + sid

    zeros16 = jnp.zeros((16,), jnp.float32)
    ones16 = jnp.ones((16,), jnp.float32)

    def zfill(i, carry):
        zero_v[pl.ds(i * 16, 16)] = zeros16
        return carry

    lax.fori_loop(0, DPT // 16, zfill, 0)

    def ofill(i, carry):
        ones_v[pl.ds(i * 16, 16)] = ones16
        return carry

    lax.fori_loop(0, CHUNK // 16, ofill, 0)

    pltpu.sync_copy(zero_v, deg_sh.at[pl.ds(sid * DPT, DPT)])
    pltpu.sync_copy(dst_hbm.at[wid], dst_v)
    plsc.subcore_barrier()

    def body(j, carry):
        pltpu.sync_copy(ones_v, deg_sh.at[dst_v.at[j]], add=True)
        return carry

    lax.fori_loop(0, JCH, body, 0)
    plsc.subcore_barrier()

    pltpu.sync_copy(
        deg_sh.at[pl.ds(sid * DPT, DPT)], out_hbm.at[cid, 0, pl.ds(sid * DPT, DPT)]
    )


@functools.partial(
    pl.kernel,
    out_type=jax.ShapeDtypeStruct((NC, NROWS, CH), jnp.float32),
    mesh=_mesh,
    scratch_types=[
        pltpu.VMEM((JCH, CHUNK), jnp.int32),
        pltpu.VMEM((JCH, CHUNK), jnp.int32),
        pltpu.VMEM((CHUNK, CH), jnp.float32),
        pltpu.SemaphoreType.DMA,
        pltpu.VMEM_SHARED((NROWS, CH), jnp.float32),
    ],
)
def _agg_kernel(g_hbm, src_hbm, dst_hbm, out_hbm, src_v, dst_v, rows_a, sem_a, acc_sh):
    cid = lax.axis_index("c")
    sid = lax.axis_index("s")
    wid = cid * NS + sid

    zeros16 = jnp.zeros((16,), jnp.float32)

    def zfill(t, carry):
        i = t // (CH // 16)
        j = t % (CH // 16)
        rows_a[i, pl.ds(j * 16, 16)] = zeros16
        return carry

    lax.fori_loop(0, CHUNK * (CH // 16), zfill, 0)

    base = sid * RPT
    for k in range(RPT // CHUNK):
        pltpu.sync_copy(rows_a, acc_sh.at[pl.ds(base + k * CHUNK, CHUNK)])
    rem = RPT % CHUNK
    if rem:
        pltpu.sync_copy(
            rows_a.at[pl.ds(0, rem)],
            acc_sh.at[pl.ds(base + (RPT // CHUNK) * CHUNK, rem)],
        )

    pltpu.sync_copy(src_hbm.at[wid], src_v)
    pltpu.sync_copy(dst_hbm.at[wid], dst_v)
    plsc.subcore_barrier()

    def body(j, carry):
        pltpu.async_copy(g_hbm.at[src_v.at[j]], rows_a, sem_a).wait()
        pltpu.sync_copy(rows_a, acc_sh.at[dst_v.at[j]], add=True)
        return carry

    lax.fori_loop(0, JCH, body, 0)
    plsc.subcore_barrier()

    pltpu.sync_copy(
        acc_sh.at[pl.ds(base, RPT)], out_hbm.at[cid, pl.ds(base, RPT)]
    )


def _dense_body(x_ref, w_ref, degp_ref, g_ref):
    w = w_ref[...]
    cn = jnp.sqrt(jnp.sum(w * w, axis=0, keepdims=True))
    scale = jnp.where(cn > 1.0, 1.0 / cn, 1.0)
    xs = x_ref[...] * scale
    h = lax.dot_general(
        xs, w, (((1,), (1,)), ((), ())), preferred_element_type=jnp.float32
    )
    dp = degp_ref[...]
    dinv = 1.0 / jnp.sqrt(dp[:, 0:1] + dp[:, 1:2] + 1.0)
    g_ref[...] = h * dinv


_dense = pl.pallas_call(
    _dense_body,
    out_shape=jax.ShapeDtypeStruct((N, CH), jnp.float32),
)


def _combine_body(p_ref, g_ref, degp_ref, b_ref, o_ref):
    dp = degp_ref[...]
    dinv = 1.0 / jnp.sqrt(dp[:, 0:1] + dp[:, 1:2] + 1.0)
    s = p_ref[0, :N, :] + p_ref[1, :N, :] + g_ref[...]
    o_ref[...] = dinv * s + b_ref[...]


_combine = pl.pallas_call(
    _combine_body,
    out_shape=jax.ShapeDtypeStruct((N, CH), jnp.float32),
)


def kernel(x, edge_index, W, b):
    e = edge_index.astype(jnp.int32)
    src = e[0]
    dst = e[1]
    pad = EPAD - E
    ar = jnp.arange(pad, dtype=jnp.int32)
    src_p = jnp.concatenate([src, ar % N])
    dst_p = jnp.concatenate([dst, TRASH + (ar % (NROWS - N))])
    src3 = src_p.reshape(NT, JCH, CHUNK)
    dst3 = dst_p.reshape(NT, JCH, CHUNK)

    deg_parts = _deg_kernel(dst3)
    degp = deg_parts.reshape(NC, DEGN).T[:N]
    g = _dense(x, W, degp)
    parts = _agg_kernel(g, src3, dst3)
    out = _combine(parts, g, degp, b.reshape(1, CH))
    return out
